# baseline (speedup 1.0000x reference)
import jax
import jax.numpy as jnp
from jax import lax
from jax.experimental import pallas as pl
from jax.experimental.pallas import tpu as pltpu
from jax.experimental.pallas import tpu_sc as plsc

A_LEVEL = (36864, 9216, 2304, 576, 144)
NW = 32
CH = (1152, 288, 80, 32, 16)
PADL = tuple(c * NW for c in CH)
OFFW = (0, 1152, 1440, 1520, 1552)
NPW = sum(CH)
NV = NPW // 16
NGT = 208
NG16 = NGT // 16
LN2 = 0.6931471805599453
SQRT2 = 1.4142135623730951


def _wid():
    return lax.axis_index("s") * 2 + lax.axis_index("c")


def _iota_i():
    return lax.broadcasted_iota(jnp.int32, (16,), 0)


def _iou(ax0, ay0, ax1, ay1, aarea, gx0, gy0, gx1, gy1, garea):
    iw = jnp.maximum(jnp.minimum(gx1, ax1) - jnp.maximum(gx0, ax0), 0.0)
    ih = jnp.maximum(jnp.minimum(gy1, ay1) - jnp.maximum(gy0, ay0), 0.0)
    inter = iw * ih
    return inter / (aarea + garea - inter + 1e-9)


def _ln(x):
    xi = plsc.bitcast(x, jnp.int32)
    e = ((xi >> 23) & 0xFF) - 127
    f = plsc.bitcast((xi & 0x7FFFFF) | 0x3F800000, jnp.float32)
    big = f > SQRT2
    f = jnp.where(big, f * 0.5, f)
    e = jnp.where(big, e + 1, e)
    t = (f - 1.0) / (f + 1.0)
    t2 = t * t
    p = 2.0 * t * (1.0 + t2 * (1.0 / 3.0 + t2 * (0.2 + t2 * (1.0 / 7.0 + t2 / 9.0))))
    return e.astype(jnp.float32) * LN2 + p


def _body_a(aw_hbm, gt_hbm, pmax_hbm, parg_hbm, a_loc, gt_loc, mx_loc, ag_loc):
    wid = _wid()
    pltpu.sync_copy(aw_hbm.at[wid], a_loc)
    pltpu.sync_copy(gt_hbm, gt_loc)
    iotaf = _iota_i().astype(jnp.float32)

    def gbody(g, _):
        g16 = g * 16
        gx0v = gt_loc[0, pl.ds(g16, 16)]
        gy0v = gt_loc[1, pl.ds(g16, 16)]
        gx1v = gt_loc[2, pl.ds(g16, 16)]
        gy1v = gt_loc[3, pl.ds(g16, 16)]
        garv = gt_loc[5, pl.ds(g16, 16)]
        mv = [jnp.zeros((16,), jnp.float32) for _ in range(5)]
        av = [jnp.zeros((16,), jnp.float32) for _ in range(5)]
        for i in range(16):
            gx0 = gx0v[i]
            gy0 = gy0v[i]
            gx1 = gx1v[i]
            gy1 = gy1v[i]
            gar = garv[i]
            for l in range(5):
                base = wid * CH[l]

                def vbody(v, carry, l=l, base=base, gx0=gx0, gy0=gy0,
                          gx1=gx1, gy1=gy1, gar=gar):
                    runm, runa = carry
                    sl = pl.ds(OFFW[l] + v * 16, 16)
                    a0 = a_loc[0, sl]
                    a1 = a_loc[1, sl]
                    a2 = a_loc[2, sl]
                    a3 = a_loc[3, sl]
                    aarea = (a2 - a0) * (a3 - a1)
                    iou = _iou(a0, a1, a2, a3, aarea, gx0, gy0, gx1, gy1, gar)
                    af = (base + v * 16).astype(jnp.float32) + iotaf
                    upd = iou > runm
                    return jnp.where(upd, iou, runm), jnp.where(upd, af, runa)

                runm = jnp.full((16,), -1.0, jnp.float32)
                runa = jnp.zeros((16,), jnp.float32)
                runm, runa = lax.fori_loop(0, CH[l] // 16, vbody, (runm, runa))
                m = jnp.max(runm, axis=0)
                am = jnp.min(jnp.where(runm == m, runa, 3e7), axis=0)
                lane = _iota_i() == i
                mv[l] = jnp.where(lane, m, mv[l])
                av[l] = jnp.where(lane, am, av[l])
        for l in range(5):
            mx_loc[l, pl.ds(g16, 16)] = mv[l]
            ag_loc[l, pl.ds(g16, 16)] = av[l]
        return 0

    lax.fori_loop(0, NG16, gbody, 0)
    for l in range(5):
        pltpu.sync_copy(mx_loc.at[pl.ds(l, 1)],
                        pmax_hbm.at[pl.ds(l * NW + wid, 1)])
        pltpu.sync_copy(ag_loc.at[pl.ds(l, 1)],
                        parg_hbm.at[pl.ds(l * NW + wid, 1)])


def _body_b(aw_hbm, gt_hbm, st_hbm, pmax_hbm, parg_hbm, labw_hbm, regw_hbm,
            a_loc, gt_loc, st_loc, stage_m, stage_a, gmax, garg, glvl,
            forced, runm8, runa8, labout, regout):
    wid = _wid()
    pltpu.sync_copy(aw_hbm.at[wid], a_loc)
    pltpu.sync_copy(gt_hbm, gt_loc)
    pltpu.sync_copy(st_hbm, st_loc)
    iotai = _iota_i()

    def mergel(l, _):
        for wb in range(NW // 8):
            pltpu.sync_copy(pmax_hbm.at[pl.ds(l * NW + wb * 8, 8)], stage_m)
            pltpu.sync_copy(parg_hbm.at[pl.ds(l * NW + wb * 8, 8)], stage_a)

            def gloop(g, _, wb=wb):
                sl = pl.ds(g * 16, 16)

                def mbody(w, carry):
                    runm, runa = carry
                    m = stage_m[w, sl]
                    a = stage_a[w, sl]
                    upd = m > runm
                    return jnp.where(upd, m, runm), jnp.where(upd, a, runa)

                if wb == 0:
                    init = (stage_m[0, sl], stage_a[0, sl])
                    lo = 1
                else:
                    init = (gmax[l, sl], garg[l, sl])
                    lo = 0
                runm, runa = lax.fori_loop(lo, 8, mbody, init)
                gmax[l, sl] = runm
                garg[l, sl] = runa
                return 0

            lax.fori_loop(0, NG16, gloop, 0)
        return 0

    lax.fori_loop(0, 5, mergel, 0)

    def lvlloop(g, _):
        sl = pl.ds(g * 16, 16)
        best = gmax[0, sl]
        lvl = jnp.zeros((16,), jnp.float32)
        for l in range(1, 5):
            m = gmax[l, sl]
            upd = m > best
            best = jnp.where(upd, m, best)
            lvl = jnp.where(upd, float(l), lvl)
        glvl[0, sl] = lvl
        return 0

    lax.fori_loop(0, NG16, lvlloop, 0)

    negv = jnp.full((16,), -1.0, jnp.float32)
    zv = jnp.zeros((16,), jnp.float32)
    for b in range(8):
        def ibody(v, _, b=b):
            forced[b, pl.ds(v * 16, 16)] = negv
            runm8[b, pl.ds(v * 16, 16)] = negv
            runa8[b, pl.ds(v * 16, 16)] = zv
            return 0
        lax.fori_loop(0, NV, ibody, 0)

    def fbody(g, _):
        sl = pl.ds(g * 16, 16)
        lvlv = glvl[0, sl]
        argv = garg[0, sl]
        chv = jnp.full((16,), CH[0], jnp.int32)
        offv = jnp.full((16,), OFFW[0], jnp.int32)
        for l in range(1, 5):
            is_l = lvlv == float(l)
            argv = jnp.where(is_l, garg[l, sl], argv)
            chv = jnp.where(is_l, CH[l], chv)
            offv = jnp.where(is_l, OFFW[l], offv)
        argi = argv.astype(jnp.int32)
        lo = wid * chv
        grpv = gt_loc[6, sl].astype(jnp.int32)
        inb = (argi >= lo) & (argi < lo + chv) & (grpv < 8)
        lidx = jnp.where(inb, argi - lo + offv, 0)
        grpc = jnp.where(inb, grpv, 0)
        jv = (g * 16 + iotai).astype(jnp.float32)
        plsc.store_scatter(forced, [grpc, lidx], jv, mask=inb)
        for _ in range(3):
            cur = plsc.load_gather(forced, [grpc, lidx])
            redo = inb & (cur < jv)
            plsc.store_scatter(forced, [grpc, lidx], jv, mask=redo)
        return 0

    lax.fori_loop(0, NG16, fbody, 0)

    def sbody(g, _):
        sl = pl.ds(g * 16, 16)
        gx0v = gt_loc[0, sl]
        gy0v = gt_loc[1, sl]
        gx1v = gt_loc[2, sl]
        gy1v = gt_loc[3, sl]
        garv = gt_loc[5, sl]
        grpv = gt_loc[6, sl].astype(jnp.int32)
        for i in range(16):
            gx0 = gx0v[i]
            gy0 = gy0v[i]
            gx1 = gx1v[i]
            gy1 = gy1v[i]
            gar = garv[i]
            bj = grpv[i]
            jf = (g * 16 + i).astype(jnp.float32)

            @pl.when(bj < 8)
            def _(bj=bj, gx0=gx0, gy0=gy0, gx1=gx1, gy1=gy1, gar=gar, jf=jf):
                def vbody(v, _):
                    asl = pl.ds(v * 16, 16)
                    a0 = a_loc[0, asl]
                    a1 = a_loc[1, asl]
                    a2 = a_loc[2, asl]
                    a3 = a_loc[3, asl]
                    aarea = (a2 - a0) * (a3 - a1)
                    iou = _iou(a0, a1, a2, a3, aarea, gx0, gy0, gx1, gy1, gar)
                    rm = runm8[bj, asl]
                    ra = runa8[bj, asl]
                    upd = iou > rm
                    runm8[bj, asl] = jnp.where(upd, iou, rm)
                    runa8[bj, asl] = jnp.where(upd, jf, ra)
                    return 0
                lax.fori_loop(0, NV, vbody, 0)
        return 0

    lax.fori_loop(0, NG16, sbody, 0)

    def wbody(v, _):
        asl = pl.ds(v * 16, 16)
        a0 = a_loc[0, asl]
        a1 = a_loc[1, asl]
        a2 = a_loc[2, asl]
        a3 = a_loc[3, asl]
        aw = a2 - a0
        ah = a3 - a1
        ax = a0 + 0.5 * aw
        ay = a1 + 0.5 * ah
        rows = jnp.zeros((16,), jnp.int32)
        for b in range(8):
            rm = runm8[b, asl]
            ra = runa8[b, asl]
            fv = forced[b, asl]
            fm = fv >= 0.0
            idx = jnp.where(fm, fv, ra).astype(jnp.int32)
            gx0 = plsc.load_gather(gt_loc, [rows, idx])
            gy0 = plsc.load_gather(gt_loc, [rows + 1, idx])
            gx1 = plsc.load_gather(gt_loc, [rows + 2, idx])
            gy1 = plsc.load_gather(gt_loc, [rows + 3, idx])
            bird = plsc.load_gather(gt_loc, [rows + 4, idx])
            pos = (rm >= 0.5) | fm
            ign = (rm >= 0.4) & (rm < 0.5)
            lab = jnp.where(pos, bird, jnp.where(ign, -1.0, 0.0))
            labout[b, asl] = lab.astype(jnp.int32)
            gw = gx1 - gx0
            gh = gy1 - gy0
            gx = gx0 + 0.5 * gw
            gy = gy0 + 0.5 * gh
            regout[b, 0, asl] = jnp.where(pos, (gx - ax) / aw, 0.0)
            regout[b, 1, asl] = jnp.where(pos, (gy - ay) / ah, 0.0)
            regout[b, 2, asl] = jnp.where(pos, _ln(gw / aw), 0.0)
            regout[b, 3, asl] = jnp.where(pos, _ln(gh / ah), 0.0)
        return 0

    lax.fori_loop(0, NV, wbody, 0)
    pltpu.sync_copy(labout, labw_hbm.at[wid])
    pltpu.sync_copy(regout, regw_hbm.at[wid])


def kernel(bb_coord, bird_ids, lengths, anchors):
    n_gt = bb_coord.shape[0]
    n_groups = len(lengths)
    mesh = plsc.VectorSubcoreMesh(core_axis_name="c", subcore_axis_name="s")

    lens = jnp.asarray(lengths, jnp.int32)
    starts = jnp.concatenate([jnp.zeros((1,), jnp.int32), jnp.cumsum(lens)])
    st_pad = jnp.zeros((16,), jnp.int32).at[: n_groups + 1].set(starts)

    jidx = jnp.arange(NGT, dtype=jnp.int32)
    grp = jnp.sum((jidx[:, None] >= starts[None, 1:]).astype(jnp.int32), axis=1)
    grp = jnp.where(jidx < n_gt, grp, 99)

    pad_gt = NGT - n_gt
    bb = jnp.concatenate([bb_coord, jnp.zeros((pad_gt, 4), jnp.float32)], axis=0)
    bid = jnp.concatenate([bird_ids.astype(jnp.float32),
                           jnp.zeros((pad_gt,), jnp.float32)], axis=0)
    area_b = (bb[:, 2] - bb[:, 0]) * (bb[:, 3] - bb[:, 1])
    zrow = jnp.zeros((NGT,), jnp.float32)
    gt8 = jnp.stack([bb[:, 0], bb[:, 1], bb[:, 2], bb[:, 3], bid, area_b,
                     grp.astype(jnp.float32), zrow], axis=0)

    pad_box = jnp.array([0.0, 0.0, 64.0, 64.0], jnp.float32)
    chunks = []
    for a, p, c in zip(anchors, PADL, CH):
        extra = p - a.shape[0]
        apad = jnp.concatenate([a, jnp.broadcast_to(pad_box, (extra, 4))], axis=0)
        chunks.append(apad.reshape(NW, c, 4))
    aw_arr = jnp.concatenate(chunks, axis=1).transpose(0, 2, 1)

    part_max, part_arg = pl.kernel(
        _body_a,
        mesh=mesh,
        compiler_params=pltpu.CompilerParams(needs_layout_passes=False),
        out_type=(
            jax.ShapeDtypeStruct((5 * NW, NGT), jnp.float32),
            jax.ShapeDtypeStruct((5 * NW, NGT), jnp.float32),
        ),
        scratch_types=[
            pltpu.VMEM((4, NPW), jnp.float32),
            pltpu.VMEM((8, NGT), jnp.float32),
            pltpu.VMEM((5, NGT), jnp.float32),
            pltpu.VMEM((5, NGT), jnp.float32),
        ],
    )(aw_arr, gt8)

    labw, regw = pl.kernel(
        _body_b,
        mesh=mesh,
        compiler_params=pltpu.CompilerParams(needs_layout_passes=False),
        out_type=(
            jax.ShapeDtypeStruct((NW, 8, NPW), jnp.int32),
            jax.ShapeDtypeStruct((NW, 8, 4, NPW), jnp.float32),
        ),
        scratch_types=[
            pltpu.VMEM((4, NPW), jnp.float32),
            pltpu.VMEM((8, NGT), jnp.float32),
            pltpu.VMEM((16,), jnp.int32),
            pltpu.VMEM((8, NGT), jnp.float32),
            pltpu.VMEM((8, NGT), jnp.float32),
            pltpu.VMEM((5, NGT), jnp.float32),
            pltpu.VMEM((5, NGT), jnp.float32),
            pltpu.VMEM((1, NGT), jnp.float32),
            pltpu.VMEM((8, NPW), jnp.float32),
            pltpu.VMEM((8, NPW), jnp.float32),
            pltpu.VMEM((8, NPW), jnp.float32),
            pltpu.VMEM((8, NPW), jnp.int32),
            pltpu.VMEM((8, 4, NPW), jnp.float32),
        ],
    )(aw_arr, gt8, st_pad, part_max, part_arg)

    labels, regs = [], []
    for a_l, c, offw in zip(A_LEVEL, CH, OFFW):
        lw = labw[:, :, offw:offw + c]
        labels.append(lw.transpose(1, 0, 2).reshape(n_groups, NW * c)[:, :a_l])
        rw = regw[:, :, :, offw:offw + c]
        regs.append(rw.transpose(1, 0, 3, 2).reshape(n_groups, NW * c, 4)[:, :a_l])
    return tuple(labels), tuple(regs)


# speedup vs baseline: 1.6195x; 1.6195x over previous
import jax
import jax.numpy as jnp
from jax import lax
from jax.experimental import pallas as pl
from jax.experimental.pallas import tpu as pltpu
from jax.experimental.pallas import tpu_sc as plsc

A_LEVEL = (36864, 9216, 2304, 576, 144)
NW = 32
CH = (1152, 288, 80, 32, 16)
PADL = tuple(c * NW for c in CH)
OFFW = (0, 1152, 1440, 1520, 1552)
NPW = sum(CH)
NV = NPW // 16
NGT = 208
NG16 = NGT // 16
LN2 = 0.6931471805599453
SQRT2 = 1.4142135623730951


def _wid():
    return lax.axis_index("s") * 2 + lax.axis_index("c")


def _iota_i():
    return lax.broadcasted_iota(jnp.int32, (16,), 0)


def _iou(ax0, ay0, ax1, ay1, aarea, gx0, gy0, gx1, gy1, garea):
    iw = jnp.maximum(jnp.minimum(gx1, ax1) - jnp.maximum(gx0, ax0), 0.0)
    ih = jnp.maximum(jnp.minimum(gy1, ay1) - jnp.maximum(gy0, ay0), 0.0)
    inter = iw * ih
    return inter / (aarea + garea - inter + 1e-9)


def _ln(x):
    xi = plsc.bitcast(x, jnp.int32)
    e = ((xi >> 23) & 0xFF) - 127
    f = plsc.bitcast((xi & 0x7FFFFF) | 0x3F800000, jnp.float32)
    big = f > SQRT2
    f = jnp.where(big, f * 0.5, f)
    e = jnp.where(big, e + 1, e)
    t = (f - 1.0) / (f + 1.0)
    t2 = t * t
    p = 2.0 * t * (1.0 + t2 * (1.0 / 3.0 + t2 * (0.2 + t2 * (1.0 / 7.0 + t2 / 9.0))))
    return e.astype(jnp.float32) * LN2 + p


def _body_a(aw_hbm, gt_hbm, pmax_hbm, parg_hbm, a_loc, gt_loc, mx_loc, ag_loc):
    wid = _wid()
    pltpu.sync_copy(aw_hbm.at[wid], a_loc)
    pltpu.sync_copy(gt_hbm, gt_loc)
    iotaf = _iota_i().astype(jnp.float32)

    def gbody(g, _):
        g16 = g * 16
        gx0v = gt_loc[0, pl.ds(g16, 16)]
        gy0v = gt_loc[1, pl.ds(g16, 16)]
        gx1v = gt_loc[2, pl.ds(g16, 16)]
        gy1v = gt_loc[3, pl.ds(g16, 16)]
        garv = gt_loc[5, pl.ds(g16, 16)]
        mv = [jnp.zeros((16,), jnp.float32) for _ in range(5)]
        av = [jnp.zeros((16,), jnp.float32) for _ in range(5)]
        for i in range(16):
            gx0 = gx0v[i]
            gy0 = gy0v[i]
            gx1 = gx1v[i]
            gy1 = gy1v[i]
            gar = garv[i]
            for l in range(5):
                base = wid * CH[l]

                def vbody(v, carry, l=l, base=base, gx0=gx0, gy0=gy0,
                          gx1=gx1, gy1=gy1, gar=gar):
                    runm, runa = carry
                    sl = pl.ds(OFFW[l] + v * 16, 16)
                    a0 = a_loc[0, sl]
                    a1 = a_loc[1, sl]
                    a2 = a_loc[2, sl]
                    a3 = a_loc[3, sl]
                    aarea = (a2 - a0) * (a3 - a1)
                    iou = _iou(a0, a1, a2, a3, aarea, gx0, gy0, gx1, gy1, gar)
                    af = (base + v * 16).astype(jnp.float32) + iotaf
                    upd = iou > runm
                    return jnp.where(upd, iou, runm), jnp.where(upd, af, runa)

                runm = jnp.full((16,), -1.0, jnp.float32)
                runa = jnp.zeros((16,), jnp.float32)
                runm, runa = lax.fori_loop(0, CH[l] // 16, vbody, (runm, runa))
                m = jnp.max(runm, axis=0)
                am = jnp.min(jnp.where(runm == m, runa, 3e7), axis=0)
                lane = _iota_i() == i
                mv[l] = jnp.where(lane, m, mv[l])
                av[l] = jnp.where(lane, am, av[l])
        for l in range(5):
            mx_loc[l, pl.ds(g16, 16)] = mv[l]
            ag_loc[l, pl.ds(g16, 16)] = av[l]
        return 0

    lax.fori_loop(0, NG16, gbody, 0)
    for l in range(5):
        pltpu.sync_copy(mx_loc.at[pl.ds(l, 1)],
                        pmax_hbm.at[pl.ds(l * NW + wid, 1)])
        pltpu.sync_copy(ag_loc.at[pl.ds(l, 1)],
                        parg_hbm.at[pl.ds(l * NW + wid, 1)])


def _body_b(aw_hbm, gt_hbm, st_hbm, pmax_hbm, parg_hbm, labw_hbm,
            regw_hbm, a_loc, gt_loc, st_loc, stage_m, stage_a, gmax,
            garg, glvl, forced, runm8, runa8, labout, regout):
    wid = _wid()
    pltpu.sync_copy(aw_hbm.at[wid], a_loc)
    pltpu.sync_copy(gt_hbm, gt_loc)
    pltpu.sync_copy(st_hbm, st_loc)
    iotai = _iota_i()
    stv = st_loc[pl.ds(0, 16)]

    def mergel(l, _):
        for wb in range(NW // 8):
            pltpu.sync_copy(pmax_hbm.at[pl.ds(l * NW + wb * 8, 8)], stage_m)
            pltpu.sync_copy(parg_hbm.at[pl.ds(l * NW + wb * 8, 8)], stage_a)

            def gloop(g, _, wb=wb):
                sl = pl.ds(g * 16, 16)

                def mbody(w, carry):
                    runm, runa = carry
                    m = stage_m[w, sl]
                    a = stage_a[w, sl]
                    upd = m > runm
                    return jnp.where(upd, m, runm), jnp.where(upd, a, runa)

                if wb == 0:
                    init = (stage_m[0, sl], stage_a[0, sl])
                    lo = 1
                else:
                    init = (gmax[l, sl], garg[l, sl])
                    lo = 0
                runm, runa = lax.fori_loop(lo, 8, mbody, init)
                gmax[l, sl] = runm
                garg[l, sl] = runa
                return 0

            lax.fori_loop(0, NG16, gloop, 0)
        return 0

    lax.fori_loop(0, 5, mergel, 0)

    def lvlloop(g, _):
        sl = pl.ds(g * 16, 16)
        best = gmax[0, sl]
        lvl = jnp.zeros((16,), jnp.float32)
        for l in range(1, 5):
            m = gmax[l, sl]
            upd = m > best
            best = jnp.where(upd, m, best)
            lvl = jnp.where(upd, float(l), lvl)
        glvl[0, sl] = lvl
        return 0

    lax.fori_loop(0, NG16, lvlloop, 0)

    negv = jnp.full((16,), -1.0, jnp.float32)
    zv = jnp.zeros((16,), jnp.float32)
    for b in range(8):
        def ibody(v, _, b=b):
            forced[b, pl.ds(v * 16, 16)] = negv
            return 0
        lax.fori_loop(0, NV, ibody, 0)

    def fbody(g, _):
        sl = pl.ds(g * 16, 16)
        lvlv = glvl[0, sl]
        argv = garg[0, sl]
        chv = jnp.full((16,), CH[0], jnp.int32)
        offv = jnp.full((16,), OFFW[0], jnp.int32)
        for l in range(1, 5):
            is_l = lvlv == float(l)
            argv = jnp.where(is_l, garg[l, sl], argv)
            chv = jnp.where(is_l, CH[l], chv)
            offv = jnp.where(is_l, OFFW[l], offv)
        argi = argv.astype(jnp.int32)
        lo = wid * chv
        grpv = gt_loc[6, sl].astype(jnp.int32)
        inb = (argi >= lo) & (argi < lo + chv) & (grpv < 8)
        lidx = jnp.where(inb, argi - lo + offv, 0)
        grpc = jnp.where(inb, grpv, 0)
        jv = (g * 16 + iotai).astype(jnp.float32)
        plsc.store_scatter(forced, [grpc, lidx], jv, mask=inb)
        for _ in range(3):
            cur = plsc.load_gather(forced, [grpc, lidx])
            redo = inb & (cur < jv)
            plsc.store_scatter(forced, [grpc, lidx], jv, mask=redo)
        return 0

    lax.fori_loop(0, NG16, fbody, 0)

    for b in range(8):
        jlo = stv[b]
        jhi = stv[b + 1]

        def vloop(v, _, jlo=jlo, jhi=jhi, b=b):
            asl = pl.ds(v * 16, 16)
            a0 = a_loc[0, asl]
            a1 = a_loc[1, asl]
            a2 = a_loc[2, asl]
            a3 = a_loc[3, asl]
            aarea = (a2 - a0) * (a3 - a1)

            def jloop(j, carry, a0=a0, a1=a1, a2=a2, a3=a3, aarea=aarea):
                runm, runa = carry
                jv = jnp.zeros((16,), jnp.int32) + j
                gx0 = plsc.load_gather(gt_loc, [iotai * 0, jv])
                gy0 = plsc.load_gather(gt_loc, [iotai * 0 + 1, jv])
                gx1 = plsc.load_gather(gt_loc, [iotai * 0 + 2, jv])
                gy1 = plsc.load_gather(gt_loc, [iotai * 0 + 3, jv])
                gar = plsc.load_gather(gt_loc, [iotai * 0 + 5, jv])
                iou = _iou(a0, a1, a2, a3, aarea, gx0, gy0, gx1, gy1, gar)
                upd = iou > runm
                runm = jnp.where(upd, iou, runm)
                runa = jnp.where(upd, j.astype(jnp.float32), runa)
                return runm, runa

            runm = jnp.full((16,), -1.0, jnp.float32)
            runa = jnp.zeros((16,), jnp.float32)
            runm, runa = lax.fori_loop(jlo, jhi, jloop, (runm, runa))
            runm8[b, asl] = runm
            runa8[b, asl] = runa
            return 0

        lax.fori_loop(0, NV, vloop, 0)

    def wbody(v, _):
        asl = pl.ds(v * 16, 16)
        a0 = a_loc[0, asl]
        a1 = a_loc[1, asl]
        a2 = a_loc[2, asl]
        a3 = a_loc[3, asl]
        aw = a2 - a0
        ah = a3 - a1
        ax = a0 + 0.5 * aw
        ay = a1 + 0.5 * ah
        rows = jnp.zeros((16,), jnp.int32)
        for b in range(8):
            rm = runm8[b, asl]
            ra = runa8[b, asl]
            fv = forced[b, asl]
            fm = fv >= 0.0
            idx = jnp.where(fm, fv, ra).astype(jnp.int32)
            gx0 = plsc.load_gather(gt_loc, [rows, idx])
            gy0 = plsc.load_gather(gt_loc, [rows + 1, idx])
            gx1 = plsc.load_gather(gt_loc, [rows + 2, idx])
            gy1 = plsc.load_gather(gt_loc, [rows + 3, idx])
            bird = plsc.load_gather(gt_loc, [rows + 4, idx])
            pos = (rm >= 0.5) | fm
            ign = (rm >= 0.4) & (rm < 0.5)
            lab = jnp.where(pos, bird, jnp.where(ign, -1.0, 0.0))
            labout[b, asl] = lab.astype(jnp.int32)
            gw = gx1 - gx0
            gh = gy1 - gy0
            gx = gx0 + 0.5 * gw
            gy = gy0 + 0.5 * gh
            regout[b, 0, asl] = jnp.where(pos, (gx - ax) / aw, 0.0)
            regout[b, 1, asl] = jnp.where(pos, (gy - ay) / ah, 0.0)
            regout[b, 2, asl] = jnp.where(pos, _ln(gw / aw), 0.0)
            regout[b, 3, asl] = jnp.where(pos, _ln(gh / ah), 0.0)
        return 0

    lax.fori_loop(0, NV, wbody, 0)
    pltpu.sync_copy(labout, labw_hbm.at[wid])
    pltpu.sync_copy(regout, regw_hbm.at[wid])


def kernel(bb_coord, bird_ids, lengths, anchors):
    n_gt = bb_coord.shape[0]
    n_groups = len(lengths)
    mesh = plsc.VectorSubcoreMesh(core_axis_name="c", subcore_axis_name="s")

    lens = jnp.asarray(lengths, jnp.int32)
    starts = jnp.concatenate([jnp.zeros((1,), jnp.int32), jnp.cumsum(lens)])
    st_pad = jnp.zeros((16,), jnp.int32).at[: n_groups + 1].set(starts)

    jidx = jnp.arange(NGT, dtype=jnp.int32)
    grp = jnp.sum((jidx[:, None] >= starts[None, 1:]).astype(jnp.int32), axis=1)
    grp = jnp.where(jidx < n_gt, grp, 99)

    pad_gt = NGT - n_gt
    bb = jnp.concatenate([bb_coord, jnp.zeros((pad_gt, 4), jnp.float32)], axis=0)
    bid = jnp.concatenate([bird_ids.astype(jnp.float32),
                           jnp.zeros((pad_gt,), jnp.float32)], axis=0)
    area_b = (bb[:, 2] - bb[:, 0]) * (bb[:, 3] - bb[:, 1])
    zrow = jnp.zeros((NGT,), jnp.float32)
    gt8 = jnp.stack([bb[:, 0], bb[:, 1], bb[:, 2], bb[:, 3], bid, area_b,
                     grp.astype(jnp.float32), zrow], axis=0)

    pad_box = jnp.array([0.0, 0.0, 64.0, 64.0], jnp.float32)
    chunks = []
    for a, p, c in zip(anchors, PADL, CH):
        extra = p - a.shape[0]
        apad = jnp.concatenate([a, jnp.broadcast_to(pad_box, (extra, 4))], axis=0)
        chunks.append(apad.reshape(NW, c, 4))
    aw_arr = jnp.concatenate(chunks, axis=1).transpose(0, 2, 1)

    part_max, part_arg = pl.kernel(
        _body_a,
        mesh=mesh,
        compiler_params=pltpu.CompilerParams(needs_layout_passes=False),
        out_type=(
            jax.ShapeDtypeStruct((5 * NW, NGT), jnp.float32),
            jax.ShapeDtypeStruct((5 * NW, NGT), jnp.float32),
        ),
        scratch_types=[
            pltpu.VMEM((4, NPW), jnp.float32),
            pltpu.VMEM((8, NGT), jnp.float32),
            pltpu.VMEM((5, NGT), jnp.float32),
            pltpu.VMEM((5, NGT), jnp.float32),
        ],
    )(aw_arr, gt8)

    labw, regw = pl.kernel(
        _body_b,
        mesh=mesh,
        compiler_params=pltpu.CompilerParams(needs_layout_passes=False),
        out_type=(
            jax.ShapeDtypeStruct((NW, 8, NPW), jnp.int32),
            jax.ShapeDtypeStruct((NW, 8, 4, NPW), jnp.float32),
        ),
        scratch_types=[
            pltpu.VMEM((4, NPW), jnp.float32),
            pltpu.VMEM((8, NGT), jnp.float32),
            pltpu.VMEM((16,), jnp.int32),
            pltpu.VMEM((8, NGT), jnp.float32),
            pltpu.VMEM((8, NGT), jnp.float32),
            pltpu.VMEM((5, NGT), jnp.float32),
            pltpu.VMEM((5, NGT), jnp.float32),
            pltpu.VMEM((1, NGT), jnp.float32),
            pltpu.VMEM((8, NPW), jnp.float32),
            pltpu.VMEM((8, NPW), jnp.float32),
            pltpu.VMEM((8, NPW), jnp.float32),
            pltpu.VMEM((8, NPW), jnp.int32),
            pltpu.VMEM((8, 4, NPW), jnp.float32),
        ],
    )(aw_arr, gt8, st_pad, part_max, part_arg)

    labels, regs = [], []
    for a_l, c, offw in zip(A_LEVEL, CH, OFFW):
        lw = labw[:, :, offw:offw + c]
        labels.append(lw.transpose(1, 0, 2).reshape(n_groups, NW * c)[:, :a_l])
        rw = regw[:, :, :, offw:offw + c]
        regs.append(rw.transpose(1, 0, 3, 2).reshape(n_groups, NW * c, 4)[:, :a_l])
    return tuple(labels), tuple(regs)
